# baseline (speedup 1.0000x reference)
import functools

import jax
import jax.numpy as jnp
from jax import lax
from jax.experimental import pallas as pl
from jax.experimental.pallas import tpu as pltpu
from jax.experimental.pallas import tpu_sc as plsc

_MAX_LEN = 2048
_D_MODEL = 1024
_BATCH = 4
_B_SC = 2

_NC = 2
_NS = 16
_NW = _NC * _NS
_LANES = 16

_ROWS_PER_W = _MAX_LEN // _NW
_CHUNK_ROWS = 16
_NCHUNK = _ROWS_PER_W // _CHUNK_ROWS
_CW = _CHUNK_ROWS * _D_MODEL
_NSTEP = _B_SC * _NCHUNK
_NBUF = 3


@functools.partial(
    pl.kernel,
    mesh=plsc.VectorSubcoreMesh(core_axis_name="c", subcore_axis_name="s"),
    out_type=jax.ShapeDtypeStruct((_B_SC, _MAX_LEN, _D_MODEL), jnp.float32),
    scratch_types=(
        [pltpu.VMEM((_ROWS_PER_W, _D_MODEL), jnp.float32)]
        + [pltpu.VMEM((_CHUNK_ROWS, _D_MODEL), jnp.float32)] * _NBUF
        + [pltpu.SemaphoreType.DMA] * (2 * _NBUF + 1)
    ),
)
def _posemb_add_sc(x_hbm, t_hbm, out_hbm, t_buf, *scratch):
    xbufs = scratch[:_NBUF]
    tsem = scratch[_NBUF]
    insems = scratch[_NBUF + 1:2 * _NBUF + 1]
    outsems = scratch[2 * _NBUF + 1:]

    wid = lax.axis_index("s") * _NC + lax.axis_index("c")
    row0 = wid * _ROWS_PER_W

    def rows(s):
        b, q = divmod(s, _NCHUNK)
        return b, pl.ds(row0 + q * _CHUNK_ROWS, _CHUNK_ROWS)

    def start_in(s):
        b, sl = rows(s)
        return pltpu.async_copy(
            x_hbm.at[b, sl, :], xbufs[s % _NBUF], insems[s % _NBUF])

    th = pltpu.async_copy(t_hbm.at[pl.ds(row0, _ROWS_PER_W), :], t_buf, tsem)
    inh = {0: start_in(0), 1: start_in(1)}
    outh = {}
    th.wait()
    for s in range(_NSTEP):
        bi = s % _NBUF
        q = s % _NCHUNK
        inh[s].wait()
        xb = xbufs[bi]

        @plsc.parallel_loop(0, _CW, step=_LANES, unroll=8)
        def _(j):
            r = jax.lax.shift_right_logical(j, 10)
            c = pl.multiple_of(jax.lax.bitwise_and(j, _D_MODEL - 1), _LANES)
            plsc.addupdate(xb.at[r, pl.ds(c, _LANES)],
                           t_buf[q * _CHUNK_ROWS + r, pl.ds(c, _LANES)])

        b, sl = rows(s)
        outh[s] = pltpu.async_copy(xb, out_hbm.at[b, sl, :], outsems[bi])
        if s + 2 < _NSTEP:
            if s >= 1:
                outh[s - 1].wait()
            inh[s + 2] = start_in(s + 2)
    for s in range(_NSTEP - _NBUF, _NSTEP):
        outh[s].wait()


_TC_ROWS = 256


def _tc_body(x_ref, t_ref, o_ref):
    o_ref[...] = x_ref[...] + t_ref[...]


_posemb_add_tc = pl.pallas_call(
    _tc_body,
    grid=(_MAX_LEN // _TC_ROWS, _BATCH - _B_SC),
    in_specs=[
        pl.BlockSpec((1, _TC_ROWS, _D_MODEL), lambda r, b: (b + _B_SC, r, 0)),
        pl.BlockSpec((_TC_ROWS, _D_MODEL), lambda r, b: (r, 0)),
    ],
    out_specs=pl.BlockSpec((1, _TC_ROWS, _D_MODEL), lambda r, b: (b, r, 0)),
    out_shape=jax.ShapeDtypeStruct(
        (_BATCH - _B_SC, _MAX_LEN, _D_MODEL), jnp.float32),
)


def kernel(x, table):
    sc_half = _posemb_add_sc(x, table)
    tc_half = _posemb_add_tc(x, table)
    return jnp.concatenate([sc_half, tc_half], axis=0)


# speedup vs baseline: 1.4314x; 1.4314x over previous
import functools

import jax
import jax.numpy as jnp
from jax import lax
from jax.experimental import pallas as pl
from jax.experimental.pallas import tpu as pltpu
from jax.experimental.pallas import tpu_sc as plsc

_MAX_LEN = 2048
_D_MODEL = 1024
_BATCH = 4

_NC = 2
_NS = 16
_NW = _NC * _NS
_LANES = 16

_ROWS_PER_W = _MAX_LEN // _NW
_CHUNK_ROWS = 16
_NCHUNK = _ROWS_PER_W // _CHUNK_ROWS
_CW = _CHUNK_ROWS * _D_MODEL
_NSTEP = _BATCH * _NCHUNK
_NBUF = 5
_LOOKAHEAD = 3


@functools.partial(
    pl.kernel,
    mesh=plsc.VectorSubcoreMesh(core_axis_name="c", subcore_axis_name="s"),
    out_type=jax.ShapeDtypeStruct((_BATCH, _MAX_LEN, _D_MODEL), jnp.float32),
    scratch_types=(
        [pltpu.VMEM((_CHUNK_ROWS, _D_MODEL), jnp.float32)] * 2
        + [pltpu.VMEM((_CHUNK_ROWS, _D_MODEL), jnp.float32)] * _NBUF
        + [pltpu.SemaphoreType.DMA] * (2 + 2 * _NBUF)
    ),
)
def _posemb_add(x_hbm, t_hbm, out_hbm, *scratch):
    tbufs = scratch[:2]
    xbufs = scratch[2:2 + _NBUF]
    tsems = scratch[2 + _NBUF:4 + _NBUF]
    insems = scratch[4 + _NBUF:4 + 2 * _NBUF]
    outsems = scratch[4 + 2 * _NBUF:]

    wid = lax.axis_index("s") * _NC + lax.axis_index("c")
    row0 = wid * _ROWS_PER_W

    def rows(s):
        q, b = divmod(s, _BATCH)
        return b, pl.ds(row0 + q * _CHUNK_ROWS, _CHUNK_ROWS)

    def start_in(s):
        b, sl = rows(s)
        return pltpu.async_copy(
            x_hbm.at[b, sl, :], xbufs[s % _NBUF], insems[s % _NBUF])

    def start_t(q):
        return pltpu.async_copy(
            t_hbm.at[pl.ds(row0 + q * _CHUNK_ROWS, _CHUNK_ROWS), :],
            tbufs[q % 2], tsems[q % 2])

    th = {0: start_t(0), 1: start_t(1)}
    inh = {s: start_in(s) for s in range(_LOOKAHEAD)}
    outh = {}
    for s in range(_NSTEP):
        bi = s % _NBUF
        q, b = divmod(s, _BATCH)
        if b == 0:
            th[q].wait()
        tb = tbufs[q % 2]
        inh[s].wait()
        xb = xbufs[bi]

        @plsc.parallel_loop(0, _CW, step=_LANES, unroll=8)
        def _(j):
            r = jax.lax.shift_right_logical(j, 10)
            c = pl.multiple_of(jax.lax.bitwise_and(j, _D_MODEL - 1), _LANES)
            plsc.addupdate(xb.at[r, pl.ds(c, _LANES)], tb[r, pl.ds(c, _LANES)])

        bb, sl = rows(s)
        outh[s] = pltpu.async_copy(xb, out_hbm.at[bb, sl, :], outsems[bi])
        if b == _BATCH - 1 and q + 2 < _NCHUNK:
            th[q + 2] = start_t(q + 2)
        if s + _LOOKAHEAD < _NSTEP:
            if s + _LOOKAHEAD >= _NBUF:
                outh[s + _LOOKAHEAD - _NBUF].wait()
            inh[s + _LOOKAHEAD] = start_in(s + _LOOKAHEAD)
    for s in range(_NSTEP - _NBUF, _NSTEP):
        outh[s].wait()


def kernel(x, table):
    return _posemb_add(x, table)
